# Initial kernel scaffold; baseline (speedup 1.0000x reference)
import jax
import jax.numpy as jnp
from jax.experimental import pallas as pl


def kernel(x, emb, W1, b1, W2, b2):
    raise NotImplementedError("write your pallas kernel here")


# speedup vs baseline: 2.2865x; 2.2865x over previous
import functools

import jax
import jax.numpy as jnp
from jax import lax
from jax.experimental import pallas as pl
from jax.experimental.pallas import tpu as pltpu
from jax.experimental.pallas import tpu_sc as plsc

B = 4096
S = 200
D = 128
SP = 208
SC_CNT = 256
HC = SP // 2
NC = 2
NS = 16
NW = NC * NS
BPW = B // NW
LD = D // 16

_mesh = plsc.VectorSubcoreMesh(
    core_axis_name="c", subcore_axis_name="s", num_cores=NC, num_subcores=NS)


@functools.partial(
    pl.kernel,
    out_type=jax.ShapeDtypeStruct((B, D), jnp.float32),
    mesh=_mesh,
    scratch_types=[
        pltpu.VMEM((BPW, 2, HC), jnp.int32),
        pltpu.VMEM((SP, D), jnp.float32),
        pltpu.VMEM((BPW, D), jnp.float32),
        pltpu.SemaphoreType.DMA,
    ],
)
def _pool(xg_hbm, emb_hbm, out_hbm, idxg_v, rows_v, sum_v, sem):
    wid = lax.axis_index("s") * NC + lax.axis_index("c")
    base = wid * BPW
    pltpu.sync_copy(xg_hbm.at[pl.ds(base, BPW)], idxg_v)

    def sample(i, carry):
        c0 = pltpu.async_copy(emb_hbm.at[idxg_v.at[i, 0]],
                              rows_v.at[pl.ds(0, HC)], sem)
        c1 = pltpu.async_copy(emb_hbm.at[idxg_v.at[i, 1]],
                              rows_v.at[pl.ds(HC, HC)], sem)
        c0.wait()
        c1.wait()

        def row_body(r, acc):
            return tuple(acc[d] + rows_v[r, pl.ds(16 * d, 16)]
                         for d in range(LD))

        acc = lax.fori_loop(
            0, SP, row_body,
            tuple(jnp.zeros((16,), jnp.float32) for _ in range(LD)))
        for d in range(LD):
            sum_v[i, pl.ds(16 * d, 16)] = acc[d]
        return carry

    lax.fori_loop(0, BPW, sample, 0)
    pltpu.sync_copy(sum_v, out_hbm.at[pl.ds(base, BPW)])


def _mlp_body(s_ref, xc_ref, w1_ref, b1_ref, w2_ref, b2_ref, o_ref):
    cnt = jnp.sum((xc_ref[...] != 0).astype(jnp.float32), axis=1,
                  keepdims=True)
    avg = s_ref[...] / jnp.maximum(cnt, 1.0)
    h = jnp.dot(avg, w1_ref[...], preferred_element_type=jnp.float32)
    h = jnp.maximum(h + b1_ref[...], 0.0)
    o_ref[...] = (jnp.dot(h, w2_ref[...], preferred_element_type=jnp.float32)
                  + b2_ref[...])


def kernel(x, emb, W1, b1, W2, b2):
    nc = W2.shape[1]
    xi = x.astype(jnp.int32)
    xg = jnp.pad(xi, ((0, 0), (0, SP - S))).reshape(B, 2, HC)
    sums = _pool(xg, emb)

    xc = jnp.pad(xi, ((0, 0), (0, SC_CNT - S)))
    w2p = jnp.zeros((D, D), W2.dtype).at[:, :nc].set(W2)
    b2p = jnp.zeros((1, D), b2.dtype).at[0, :nc].set(b2)
    out = pl.pallas_call(
        _mlp_body,
        out_shape=jax.ShapeDtypeStruct((B, D), jnp.float32),
    )(sums, xc, W1, b1.reshape(1, D), w2p, b2p)
    return out[:, :nc]
